# Initial kernel scaffold; baseline (speedup 1.0000x reference)
import functools

import jax
import jax.numpy as jnp
from jax import lax
from jax.experimental import pallas as pl
from jax.experimental.pallas import tpu as pltpu

D = 768; H = 12; F = 2048; E = 8; KTOP = 2; HD = 64
EPS = 1e-5
S = 2048; MEM = 2048
T = S
NPAIR = T * KTOP
BLK = 256
PAD = 6144
NB = PAD // BLK

_INTERPRET = False


def _dot(a, b, dims):
    return lax.dot_general(a, b, (dims, ((), ())),
                           preferred_element_type=jnp.float32)


def _matmul_bias(x, w, b, bn=768):
    t, k = x.shape
    n = w.shape[0]

    def body(x_ref, w_ref, b_ref, o_ref):
        o_ref[...] = _dot(x_ref[...], w_ref[...], ((1,), (1,))) + b_ref[...]

    return pl.pallas_call(
        body,
        grid=(n // bn,),
        in_specs=[
            pl.BlockSpec((t, k), lambda j: (0, 0)),
            pl.BlockSpec((bn, k), lambda j: (j, 0)),
            pl.BlockSpec((1, bn), lambda j: (0, j)),
        ],
        out_specs=pl.BlockSpec((t, bn), lambda j: (0, j)),
        out_shape=jax.ShapeDtypeStruct((t, n), jnp.float32),
        interpret=_INTERPRET,
    )(x, w, b.reshape(1, n))


def _attention(q, kv, bq=512):
    m = kv.shape[0]

    def body(q_ref, k_ref, v_ref, o_ref):
        s = _dot(q_ref[...], k_ref[...], ((1,), (1,))) * (1.0 / 8.0)
        mx = jnp.max(s, axis=-1, keepdims=True)
        p = jnp.exp(s - mx)
        l = jnp.sum(p, axis=-1, keepdims=True)
        o = _dot(p, v_ref[...], ((1,), (0,)))
        o_ref[...] = o / l

    return pl.pallas_call(
        body,
        grid=(H, S // bq),
        in_specs=[
            pl.BlockSpec((bq, HD), lambda h, i: (i, h)),
            pl.BlockSpec((m, HD), lambda h, i: (0, h)),
            pl.BlockSpec((m, HD), lambda h, i: (0, H + h)),
        ],
        out_specs=pl.BlockSpec((bq, HD), lambda h, i: (i, h)),
        out_shape=jax.ShapeDtypeStruct((S, D), jnp.float32),
        interpret=_INTERPRET,
    )(q, kv, kv)


def _proj_res_ln(ctx, w_out, b_out, resid, g, b, bm=512):
    def body(c_ref, w_ref, bo_ref, r_ref, g_ref, b_ref, o_ref):
        o = _dot(c_ref[...], w_ref[...], ((1,), (1,))) + bo_ref[...]
        z = r_ref[...] + o
        mu = jnp.mean(z, axis=-1, keepdims=True)
        zc = z - mu
        var = jnp.mean(zc * zc, axis=-1, keepdims=True)
        o_ref[...] = zc * lax.rsqrt(var + EPS) * g_ref[...] + b_ref[...]

    return pl.pallas_call(
        body,
        grid=(S // bm,),
        in_specs=[
            pl.BlockSpec((bm, D), lambda i: (i, 0)),
            pl.BlockSpec((D, D), lambda i: (0, 0)),
            pl.BlockSpec((1, D), lambda i: (0, 0)),
            pl.BlockSpec((bm, D), lambda i: (i, 0)),
            pl.BlockSpec((1, D), lambda i: (0, 0)),
            pl.BlockSpec((1, D), lambda i: (0, 0)),
        ],
        out_specs=pl.BlockSpec((bm, D), lambda i: (i, 0)),
        out_shape=jax.ShapeDtypeStruct((S, D), jnp.float32),
        interpret=_INTERPRET,
    )(ctx, w_out, b_out.reshape(1, D), resid, g.reshape(1, D), b.reshape(1, D))


def _router(x, rw, rb):
    def body(x_ref, rw_ref, rb_ref, eid_ref, gate_ref, aux_ref):
        logits = _dot(x_ref[...], rw_ref[...], ((1,), (1,))) + rb_ref[...]
        mx = jnp.max(logits, axis=-1, keepdims=True)
        ex = jnp.exp(logits - mx)
        p = ex / jnp.sum(ex, axis=-1, keepdims=True)
        iot = lax.broadcasted_iota(jnp.int32, (T, E), 1)
        m1 = jnp.max(p, axis=-1, keepdims=True)
        i1 = jnp.min(jnp.where(p == m1, iot, E), axis=-1, keepdims=True)
        pm = jnp.where(iot == i1, -1.0, p)
        m2 = jnp.max(pm, axis=-1, keepdims=True)
        i2 = jnp.min(jnp.where(pm == m2, iot, E), axis=-1, keepdims=True)
        gs = m1 + m2
        eid_ref[...] = jnp.concatenate([i1, i2], axis=1)
        gate_ref[...] = jnp.concatenate([m1 / gs, m2 / gs], axis=1)
        oh = ((iot == i1) | (iot == i2)).astype(jnp.float32)
        frac = jnp.sum(oh, axis=0, keepdims=True) / (T * KTOP)
        imp = jnp.sum(p, axis=0, keepdims=True) / T
        aux_ref[...] = jnp.float32(E) * jnp.sum(frac * imp).reshape(1, 1)

    return pl.pallas_call(
        body,
        in_specs=[
            pl.BlockSpec((T, D), lambda: (0, 0)),
            pl.BlockSpec((E, D), lambda: (0, 0)),
            pl.BlockSpec((1, E), lambda: (0, 0)),
        ],
        out_specs=[
            pl.BlockSpec((T, 2), lambda: (0, 0)),
            pl.BlockSpec((T, 2), lambda: (0, 0)),
            pl.BlockSpec((1, 1), lambda: (0, 0)),
        ],
        out_shape=[
            jax.ShapeDtypeStruct((T, 2), jnp.int32),
            jax.ShapeDtypeStruct((T, 2), jnp.float32),
            jax.ShapeDtypeStruct((1, 1), jnp.float32),
        ],
        interpret=_INTERPRET,
    )(x, rw, rb.reshape(1, E))


def _moe_ffn(block_expert, xg, w1, b1, w2, b2):
    def body(be_ref, xg_ref, w1_ref, b1_ref, w2_ref, b2_ref, y_ref):
        i = pl.program_id(0)

        @pl.when(be_ref[i] >= 0)
        def _():
            h = _dot(xg_ref[...], w1_ref[0], ((1,), (1,))) + b1_ref[...]
            h = jnp.maximum(h, 0.0)
            y_ref[...] = _dot(h, w2_ref[0], ((1,), (1,))) + b2_ref[...]

    def _e(i, be_ref):
        return jnp.maximum(be_ref[i], 0)

    grid_spec = pltpu.PrefetchScalarGridSpec(
        num_scalar_prefetch=1,
        grid=(NB,),
        in_specs=[
            pl.BlockSpec((BLK, D), lambda i, be: (i, 0)),
            pl.BlockSpec((1, F, D), lambda i, be: (_e(i, be), 0, 0)),
            pl.BlockSpec((1, F), lambda i, be: (_e(i, be), 0)),
            pl.BlockSpec((1, D, F), lambda i, be: (_e(i, be), 0, 0)),
            pl.BlockSpec((1, D), lambda i, be: (_e(i, be), 0)),
        ],
        out_specs=pl.BlockSpec((BLK, D), lambda i, be: (i, 0)),
    )
    return pl.pallas_call(
        body,
        grid_spec=grid_spec,
        out_shape=jax.ShapeDtypeStruct((PAD, D), jnp.float32),
        interpret=_INTERPRET,
    )(block_expert, xg, w1, b1, w2, b2)


def _combine_ln(x, y0, y1, gates, g, b, bm=512):
    def body(x_ref, y0_ref, y1_ref, gt_ref, g_ref, b_ref, o_ref):
        gt = gt_ref[...]
        mo = gt[:, 0:1] * y0_ref[...] + gt[:, 1:2] * y1_ref[...]
        z = x_ref[...] + mo
        mu = jnp.mean(z, axis=-1, keepdims=True)
        zc = z - mu
        var = jnp.mean(zc * zc, axis=-1, keepdims=True)
        o_ref[...] = zc * lax.rsqrt(var + EPS) * g_ref[...] + b_ref[...]

    return pl.pallas_call(
        body,
        grid=(S // bm,),
        in_specs=[
            pl.BlockSpec((bm, D), lambda i: (i, 0)),
            pl.BlockSpec((bm, D), lambda i: (i, 0)),
            pl.BlockSpec((bm, D), lambda i: (i, 0)),
            pl.BlockSpec((bm, 2), lambda i: (i, 0)),
            pl.BlockSpec((1, D), lambda i: (0, 0)),
            pl.BlockSpec((1, D), lambda i: (0, 0)),
        ],
        out_specs=pl.BlockSpec((bm, D), lambda i: (i, 0)),
        out_shape=jax.ShapeDtypeStruct((S, D), jnp.float32),
        interpret=_INTERPRET,
    )(x, y0, y1, gates, g.reshape(1, D), b.reshape(1, D))


def _dispatch_jnp(eids):
    ef = eids.T.reshape(NPAIR)
    earange = jnp.arange(E, dtype=jnp.int32)
    counts = jnp.sum((ef[:, None] == earange[None, :]).astype(jnp.int32), axis=0)
    padded = ((counts + BLK - 1) // BLK) * BLK
    off = jnp.concatenate([jnp.zeros((1,), jnp.int32), jnp.cumsum(padded)])[:E]
    rank = jnp.zeros((NPAIR,), jnp.int32)
    for e in range(E):
        msk = ef == e
        csum = jnp.cumsum(msk.astype(jnp.int32)) - 1
        rank = jnp.where(msk, csum, rank)
    pos = off[ef] + rank
    perm = jnp.zeros((PAD,), jnp.int32).at[pos].set(
        jnp.arange(NPAIR, dtype=jnp.int32) % T)
    be = jnp.full((NB,), -1, jnp.int32)
    blk_ids = jnp.arange(NB, dtype=jnp.int32) * BLK
    for e in range(E):
        msk = (blk_ids >= off[e]) & (blk_ids < off[e] + padded[e])
        be = jnp.where(msk, e, be)
    return pos, perm, be


def kernel(tgt, memory, sa_w_in, sa_b_in, sa_w_out, sa_b_out,
           ca_w_in, ca_b_in, ca_w_out, ca_b_out,
           ln1_g, ln1_b, ln2_g, ln2_b, ln3_g, ln3_b,
           router_w, router_b, w1, b1, w2, b2):
    x0 = tgt[0]
    mem = memory[0]

    qkv = _matmul_bias(x0, sa_w_in, sa_b_in)
    ctx = _attention(qkv[:, :D], qkv[:, D:])
    x1 = _proj_res_ln(ctx, sa_w_out, sa_b_out, x0, ln1_g, ln1_b)

    q = _matmul_bias(x1, ca_w_in[:D], ca_b_in[:D])
    kv = _matmul_bias(mem, ca_w_in[D:], ca_b_in[D:])
    ctx2 = _attention(q, kv)
    x2 = _proj_res_ln(ctx2, ca_w_out, ca_b_out, x1, ln2_g, ln2_b)

    eids, gates, aux = _router(x2, router_w, router_b)
    pos, perm, be = _dispatch_jnp(eids)
    xg = jnp.take(x2, perm, axis=0)
    y = _moe_ffn(be, xg, w1, b1, w2, b2)
    y0 = jnp.take(y, pos[:T], axis=0)
    y1 = jnp.take(y, pos[T:], axis=0)
    xo = _combine_ln(x2, y0, y1, gates, ln3_g, ln3_b)

    return (xo[None], aux.reshape(()))


# speedup vs baseline: 1.2077x; 1.2077x over previous
import functools

import jax
import jax.numpy as jnp
from jax import lax
from jax.experimental import pallas as pl
from jax.experimental.pallas import tpu as pltpu

D = 768; H = 12; F = 2048; E = 8; KTOP = 2; HD = 64
EPS = 1e-5
S = 2048; MEM = 2048
T = S
NPAIR = T * KTOP
BLK = 256
PAD = 6144
NB = PAD // BLK

_INTERPRET = False


def _dot(a, b, dims):
    return lax.dot_general(a, b, (dims, ((), ())),
                           preferred_element_type=jnp.float32)


def _matmul_bias(x, w, b, bn=768):
    t, k = x.shape
    n = w.shape[0]

    def body(x_ref, w_ref, b_ref, o_ref):
        o_ref[...] = _dot(x_ref[...], w_ref[...], ((1,), (1,))) + b_ref[...]

    return pl.pallas_call(
        body,
        grid=(n // bn,),
        in_specs=[
            pl.BlockSpec((t, k), lambda j: (0, 0)),
            pl.BlockSpec((bn, k), lambda j: (j, 0)),
            pl.BlockSpec((1, bn), lambda j: (0, j)),
        ],
        out_specs=pl.BlockSpec((t, bn), lambda j: (0, j)),
        out_shape=jax.ShapeDtypeStruct((t, n), jnp.float32),
        interpret=_INTERPRET,
    )(x, w, b.reshape(1, n))


def _attention(q, kv, bq=512):
    m = kv.shape[0]

    def one_head(qh, kh, vh):
        s = _dot(qh, kh, ((1,), (1,))) * (1.0 / 8.0)
        mx = jnp.max(s, axis=-1, keepdims=True)
        p = jnp.exp(s - mx)
        l = jnp.sum(p, axis=-1, keepdims=True)
        return _dot(p, vh, ((1,), (0,))) / l

    def body(q_ref, k_ref, v_ref, o_ref):
        qb, kb, vb = q_ref[...], k_ref[...], v_ref[...]
        o_ref[...] = jnp.concatenate(
            [one_head(qb[:, :HD], kb[:, :HD], vb[:, :HD]),
             one_head(qb[:, HD:], kb[:, HD:], vb[:, HD:])], axis=1)

    return pl.pallas_call(
        body,
        grid=(H // 2, S // bq),
        in_specs=[
            pl.BlockSpec((bq, 2 * HD), lambda h, i: (i, h)),
            pl.BlockSpec((m, 2 * HD), lambda h, i: (0, h)),
            pl.BlockSpec((m, 2 * HD), lambda h, i: (0, H // 2 + h)),
        ],
        out_specs=pl.BlockSpec((bq, 2 * HD), lambda h, i: (i, h)),
        out_shape=jax.ShapeDtypeStruct((S, D), jnp.float32),
        interpret=_INTERPRET,
    )(q, kv, kv)


def _proj_res_ln(ctx, w_out, b_out, resid, g, b, bm=512):
    def body(c_ref, w_ref, bo_ref, r_ref, g_ref, b_ref, o_ref):
        o = _dot(c_ref[...], w_ref[...], ((1,), (1,))) + bo_ref[...]
        z = r_ref[...] + o
        mu = jnp.mean(z, axis=-1, keepdims=True)
        zc = z - mu
        var = jnp.mean(zc * zc, axis=-1, keepdims=True)
        o_ref[...] = zc * lax.rsqrt(var + EPS) * g_ref[...] + b_ref[...]

    return pl.pallas_call(
        body,
        grid=(S // bm,),
        in_specs=[
            pl.BlockSpec((bm, D), lambda i: (i, 0)),
            pl.BlockSpec((D, D), lambda i: (0, 0)),
            pl.BlockSpec((1, D), lambda i: (0, 0)),
            pl.BlockSpec((bm, D), lambda i: (i, 0)),
            pl.BlockSpec((1, D), lambda i: (0, 0)),
            pl.BlockSpec((1, D), lambda i: (0, 0)),
        ],
        out_specs=pl.BlockSpec((bm, D), lambda i: (i, 0)),
        out_shape=jax.ShapeDtypeStruct((S, D), jnp.float32),
        interpret=_INTERPRET,
    )(ctx, w_out, b_out.reshape(1, D), resid, g.reshape(1, D), b.reshape(1, D))


def _router(x, rw, rb):
    def body(x_ref, rw_ref, rb_ref, eid_ref, gate_ref, aux_ref):
        logits = _dot(x_ref[...], rw_ref[...], ((1,), (1,))) + rb_ref[...]
        mx = jnp.max(logits, axis=-1, keepdims=True)
        ex = jnp.exp(logits - mx)
        p = ex / jnp.sum(ex, axis=-1, keepdims=True)
        iot = lax.broadcasted_iota(jnp.int32, (T, E), 1)
        m1 = jnp.max(p, axis=-1, keepdims=True)
        i1 = jnp.min(jnp.where(p == m1, iot, E), axis=-1, keepdims=True)
        pm = jnp.where(iot == i1, -1.0, p)
        m2 = jnp.max(pm, axis=-1, keepdims=True)
        i2 = jnp.min(jnp.where(pm == m2, iot, E), axis=-1, keepdims=True)
        gs = m1 + m2
        eid_ref[...] = jnp.concatenate([i1, i2], axis=1)
        gate_ref[...] = jnp.concatenate([m1 / gs, m2 / gs], axis=1)
        oh = ((iot == i1) | (iot == i2)).astype(jnp.float32)
        frac = jnp.sum(oh, axis=0, keepdims=True) / (T * KTOP)
        imp = jnp.sum(p, axis=0, keepdims=True) / T
        aux_ref[...] = jnp.float32(E) * jnp.sum(frac * imp).reshape(1, 1)

    return pl.pallas_call(
        body,
        in_specs=[
            pl.BlockSpec((T, D), lambda: (0, 0)),
            pl.BlockSpec((E, D), lambda: (0, 0)),
            pl.BlockSpec((1, E), lambda: (0, 0)),
        ],
        out_specs=[
            pl.BlockSpec((T, 2), lambda: (0, 0)),
            pl.BlockSpec((T, 2), lambda: (0, 0)),
            pl.BlockSpec((1, 1), lambda: (0, 0)),
        ],
        out_shape=[
            jax.ShapeDtypeStruct((T, 2), jnp.int32),
            jax.ShapeDtypeStruct((T, 2), jnp.float32),
            jax.ShapeDtypeStruct((1, 1), jnp.float32),
        ],
        interpret=_INTERPRET,
    )(x, rw, rb.reshape(1, E))


def _moe_ffn(block_expert, xg, w1, b1, w2, b2):
    def body(be_ref, xg_ref, w1_ref, b1_ref, w2_ref, b2_ref, y_ref):
        i = pl.program_id(0)

        @pl.when(be_ref[i] >= 0)
        def _():
            h = _dot(xg_ref[...], w1_ref[0], ((1,), (1,))) + b1_ref[0]
            h = jnp.maximum(h, 0.0)
            y_ref[...] = _dot(h, w2_ref[0], ((1,), (1,))) + b2_ref[0]

    def _e(i, be_ref):
        return jnp.maximum(be_ref[i], 0)

    grid_spec = pltpu.PrefetchScalarGridSpec(
        num_scalar_prefetch=1,
        grid=(NB,),
        in_specs=[
            pl.BlockSpec((BLK, D), lambda i, be: (i, 0)),
            pl.BlockSpec((1, F, D), lambda i, be: (_e(i, be), 0, 0)),
            pl.BlockSpec((1, 1, F), lambda i, be: (_e(i, be), 0, 0)),
            pl.BlockSpec((1, D, F), lambda i, be: (_e(i, be), 0, 0)),
            pl.BlockSpec((1, 1, D), lambda i, be: (_e(i, be), 0, 0)),
        ],
        out_specs=pl.BlockSpec((BLK, D), lambda i, be: (i, 0)),
    )
    return pl.pallas_call(
        body,
        grid_spec=grid_spec,
        out_shape=jax.ShapeDtypeStruct((PAD, D), jnp.float32),
        interpret=_INTERPRET,
    )(block_expert, xg, w1, b1.reshape(E, 1, F), w2, b2.reshape(E, 1, D))


def _combine_ln(x, y0, y1, gates, g, b, bm=512):
    def body(x_ref, y0_ref, y1_ref, gt_ref, g_ref, b_ref, o_ref):
        gt = gt_ref[...]
        mo = gt[:, 0:1] * y0_ref[...] + gt[:, 1:2] * y1_ref[...]
        z = x_ref[...] + mo
        mu = jnp.mean(z, axis=-1, keepdims=True)
        zc = z - mu
        var = jnp.mean(zc * zc, axis=-1, keepdims=True)
        o_ref[...] = zc * lax.rsqrt(var + EPS) * g_ref[...] + b_ref[...]

    return pl.pallas_call(
        body,
        grid=(S // bm,),
        in_specs=[
            pl.BlockSpec((bm, D), lambda i: (i, 0)),
            pl.BlockSpec((bm, D), lambda i: (i, 0)),
            pl.BlockSpec((bm, D), lambda i: (i, 0)),
            pl.BlockSpec((bm, 2), lambda i: (i, 0)),
            pl.BlockSpec((1, D), lambda i: (0, 0)),
            pl.BlockSpec((1, D), lambda i: (0, 0)),
        ],
        out_specs=pl.BlockSpec((bm, D), lambda i: (i, 0)),
        out_shape=jax.ShapeDtypeStruct((S, D), jnp.float32),
        interpret=_INTERPRET,
    )(x, y0, y1, gates, g.reshape(1, D), b.reshape(1, D))


def _dispatch_jnp(eids):
    ef = eids.T.reshape(NPAIR)
    earange = jnp.arange(E, dtype=jnp.int32)
    counts = jnp.sum((ef[:, None] == earange[None, :]).astype(jnp.int32), axis=0)
    padded = ((counts + BLK - 1) // BLK) * BLK
    off = jnp.concatenate([jnp.zeros((1,), jnp.int32), jnp.cumsum(padded)])[:E]
    rank = jnp.zeros((NPAIR,), jnp.int32)
    for e in range(E):
        msk = ef == e
        csum = jnp.cumsum(msk.astype(jnp.int32)) - 1
        rank = jnp.where(msk, csum, rank)
    pos = off[ef] + rank
    perm = jnp.zeros((PAD,), jnp.int32).at[pos].set(
        jnp.arange(NPAIR, dtype=jnp.int32) % T)
    be = jnp.full((NB,), -1, jnp.int32)
    blk_ids = jnp.arange(NB, dtype=jnp.int32) * BLK
    for e in range(E):
        msk = (blk_ids >= off[e]) & (blk_ids < off[e] + padded[e])
        be = jnp.where(msk, e, be)
    return pos, perm, be


def kernel(tgt, memory, sa_w_in, sa_b_in, sa_w_out, sa_b_out,
           ca_w_in, ca_b_in, ca_w_out, ca_b_out,
           ln1_g, ln1_b, ln2_g, ln2_b, ln3_g, ln3_b,
           router_w, router_b, w1, b1, w2, b2):
    x0 = tgt[0]
    mem = memory[0]

    qkv = _matmul_bias(x0, sa_w_in, sa_b_in)
    ctx = _attention(qkv[:, :D], qkv[:, D:])
    x1 = _proj_res_ln(ctx, sa_w_out, sa_b_out, x0, ln1_g, ln1_b)

    q = _matmul_bias(x1, ca_w_in[:D], ca_b_in[:D])
    kv = _matmul_bias(mem, ca_w_in[D:], ca_b_in[D:])
    ctx2 = _attention(q, kv)
    x2 = _proj_res_ln(ctx2, ca_w_out, ca_b_out, x1, ln2_g, ln2_b)

    eids, gates, aux = _router(x2, router_w, router_b)
    pos, perm, be = _dispatch_jnp(eids)
    xg = jnp.take(x2, perm, axis=0)
    y = _moe_ffn(be, xg, w1, b1, w2, b2)
    y0 = jnp.take(y, pos[:T], axis=0)
    y1 = jnp.take(y, pos[T:], axis=0)
    xo = _combine_ln(x2, y0, y1, gates, ln3_g, ln3_b)

    return (xo[None], aux.reshape(()))


# speedup vs baseline: 1.4728x; 1.2196x over previous
import functools

import jax
import jax.numpy as jnp
from jax import lax
from jax.experimental import pallas as pl
from jax.experimental.pallas import tpu as pltpu
from jax.experimental.pallas import tpu_sc as plsc

D = 768; H = 12; F = 2048; E = 8; KTOP = 2; HD = 64
EPS = 1e-5
S = 2048; MEM = 2048
T = S
NPAIR = T * KTOP
BLK = 256
PAD = 6144
NB = PAD // BLK

_INTERPRET = False


def _dot(a, b, dims):
    return lax.dot_general(a, b, (dims, ((), ())),
                           preferred_element_type=jnp.float32)


def _matmul_bias(x, w, b, bn=768):
    t, k = x.shape
    n = w.shape[0]

    def body(x_ref, w_ref, b_ref, o_ref):
        o_ref[...] = _dot(x_ref[...], w_ref[...], ((1,), (1,))) + b_ref[...]

    return pl.pallas_call(
        body,
        grid=(n // bn,),
        in_specs=[
            pl.BlockSpec((t, k), lambda j: (0, 0)),
            pl.BlockSpec((bn, k), lambda j: (j, 0)),
            pl.BlockSpec((1, bn), lambda j: (0, j)),
        ],
        out_specs=pl.BlockSpec((t, bn), lambda j: (0, j)),
        out_shape=jax.ShapeDtypeStruct((t, n), jnp.float32),
        interpret=_INTERPRET,
    )(x, w, b.reshape(1, n))


def _attention(q, kv, bq=512):
    m = kv.shape[0]

    def one_head(qh, kh, vh):
        s = _dot(qh, kh, ((1,), (1,))) * (1.0 / 8.0)
        mx = jnp.max(s, axis=-1, keepdims=True)
        p = jnp.exp(s - mx)
        l = jnp.sum(p, axis=-1, keepdims=True)
        return _dot(p, vh, ((1,), (0,))) / l

    def body(q_ref, k_ref, v_ref, o_ref):
        qb, kb, vb = q_ref[...], k_ref[...], v_ref[...]
        o_ref[...] = jnp.concatenate(
            [one_head(qb[:, :HD], kb[:, :HD], vb[:, :HD]),
             one_head(qb[:, HD:], kb[:, HD:], vb[:, HD:])], axis=1)

    return pl.pallas_call(
        body,
        grid=(H // 2, S // bq),
        in_specs=[
            pl.BlockSpec((bq, 2 * HD), lambda h, i: (i, h)),
            pl.BlockSpec((m, 2 * HD), lambda h, i: (0, h)),
            pl.BlockSpec((m, 2 * HD), lambda h, i: (0, H // 2 + h)),
        ],
        out_specs=pl.BlockSpec((bq, 2 * HD), lambda h, i: (i, h)),
        out_shape=jax.ShapeDtypeStruct((S, D), jnp.float32),
        interpret=_INTERPRET,
    )(q, kv, kv)


def _proj_res_ln(ctx, w_out, b_out, resid, g, b, bm=512):
    def body(c_ref, w_ref, bo_ref, r_ref, g_ref, b_ref, o_ref):
        o = _dot(c_ref[...], w_ref[...], ((1,), (1,))) + bo_ref[...]
        z = r_ref[...] + o
        mu = jnp.mean(z, axis=-1, keepdims=True)
        zc = z - mu
        var = jnp.mean(zc * zc, axis=-1, keepdims=True)
        o_ref[...] = zc * lax.rsqrt(var + EPS) * g_ref[...] + b_ref[...]

    return pl.pallas_call(
        body,
        grid=(S // bm,),
        in_specs=[
            pl.BlockSpec((bm, D), lambda i: (i, 0)),
            pl.BlockSpec((D, D), lambda i: (0, 0)),
            pl.BlockSpec((1, D), lambda i: (0, 0)),
            pl.BlockSpec((bm, D), lambda i: (i, 0)),
            pl.BlockSpec((1, D), lambda i: (0, 0)),
            pl.BlockSpec((1, D), lambda i: (0, 0)),
        ],
        out_specs=pl.BlockSpec((bm, D), lambda i: (i, 0)),
        out_shape=jax.ShapeDtypeStruct((S, D), jnp.float32),
        interpret=_INTERPRET,
    )(ctx, w_out, b_out.reshape(1, D), resid, g.reshape(1, D), b.reshape(1, D))


def _router(x, rw, rb):
    def body(x_ref, rw_ref, rb_ref, eid_ref, gate_ref, aux_ref):
        logits = _dot(x_ref[...], rw_ref[...], ((1,), (1,))) + rb_ref[...]
        mx = jnp.max(logits, axis=-1, keepdims=True)
        ex = jnp.exp(logits - mx)
        p = ex / jnp.sum(ex, axis=-1, keepdims=True)
        iot = lax.broadcasted_iota(jnp.int32, (T, E), 1)
        m1 = jnp.max(p, axis=-1, keepdims=True)
        i1 = jnp.min(jnp.where(p == m1, iot, E), axis=-1, keepdims=True)
        pm = jnp.where(iot == i1, -1.0, p)
        m2 = jnp.max(pm, axis=-1, keepdims=True)
        i2 = jnp.min(jnp.where(pm == m2, iot, E), axis=-1, keepdims=True)
        gs = m1 + m2
        eid_ref[...] = jnp.concatenate([i1, i2], axis=1)
        gate_ref[...] = jnp.concatenate([m1 / gs, m2 / gs], axis=1)
        oh = ((iot == i1) | (iot == i2)).astype(jnp.float32)
        frac = jnp.sum(oh, axis=0, keepdims=True) / (T * KTOP)
        imp = jnp.sum(p, axis=0, keepdims=True) / T
        aux_ref[...] = jnp.float32(E) * jnp.sum(frac * imp).reshape(1, 1)

    return pl.pallas_call(
        body,
        in_specs=[
            pl.BlockSpec((T, D), lambda: (0, 0)),
            pl.BlockSpec((E, D), lambda: (0, 0)),
            pl.BlockSpec((1, E), lambda: (0, 0)),
        ],
        out_specs=[
            pl.BlockSpec((T, 2), lambda: (0, 0)),
            pl.BlockSpec((T, 2), lambda: (0, 0)),
            pl.BlockSpec((1, 1), lambda: (0, 0)),
        ],
        out_shape=[
            jax.ShapeDtypeStruct((T, 2), jnp.int32),
            jax.ShapeDtypeStruct((T, 2), jnp.float32),
            jax.ShapeDtypeStruct((1, 1), jnp.float32),
        ],
        interpret=_INTERPRET,
    )(x, rw, rb.reshape(1, E))


def _moe_ffn(block_expert, xg, w1, b1, w2, b2):
    def body(be_ref, xg_ref, w1_ref, b1_ref, w2_ref, b2_ref, y_ref):
        i = pl.program_id(0)

        @pl.when(be_ref[i] >= 0)
        def _():
            h = _dot(xg_ref[...], w1_ref[0], ((1,), (1,))) + b1_ref[0]
            h = jnp.maximum(h, 0.0)
            y_ref[...] = _dot(h, w2_ref[0], ((1,), (1,))) + b2_ref[0]

    def _e(i, be_ref):
        return jnp.maximum(be_ref[i], 0)

    grid_spec = pltpu.PrefetchScalarGridSpec(
        num_scalar_prefetch=1,
        grid=(NB,),
        in_specs=[
            pl.BlockSpec((BLK, D), lambda i, be: (i, 0)),
            pl.BlockSpec((1, F, D), lambda i, be: (_e(i, be), 0, 0)),
            pl.BlockSpec((1, 1, F), lambda i, be: (_e(i, be), 0, 0)),
            pl.BlockSpec((1, D, F), lambda i, be: (_e(i, be), 0, 0)),
            pl.BlockSpec((1, 1, D), lambda i, be: (_e(i, be), 0, 0)),
        ],
        out_specs=pl.BlockSpec((BLK, D), lambda i, be: (i, 0)),
    )
    return pl.pallas_call(
        body,
        grid_spec=grid_spec,
        out_shape=jax.ShapeDtypeStruct((PAD, D), jnp.float32),
        interpret=_INTERPRET,
    )(block_expert, xg, w1, b1.reshape(E, 1, F), w2, b2.reshape(E, 1, D))


def _combine_ln(x, y01, gates, g, b, bm=512):
    def body(x_ref, y0_ref, y1_ref, gt_ref, g_ref, b_ref, o_ref):
        gt = gt_ref[...]
        mo = gt[:, 0:1] * y0_ref[...] + gt[:, 1:2] * y1_ref[...]
        z = x_ref[...] + mo
        mu = jnp.mean(z, axis=-1, keepdims=True)
        zc = z - mu
        var = jnp.mean(zc * zc, axis=-1, keepdims=True)
        o_ref[...] = zc * lax.rsqrt(var + EPS) * g_ref[...] + b_ref[...]

    return pl.pallas_call(
        body,
        grid=(S // bm,),
        in_specs=[
            pl.BlockSpec((bm, D), lambda i: (i, 0)),
            pl.BlockSpec((bm, D), lambda i: (i, 0)),
            pl.BlockSpec((bm, D), lambda i: (i + T // bm, 0)),
            pl.BlockSpec((bm, 2), lambda i: (i, 0)),
            pl.BlockSpec((1, D), lambda i: (0, 0)),
            pl.BlockSpec((1, D), lambda i: (0, 0)),
        ],
        out_specs=pl.BlockSpec((bm, D), lambda i: (i, 0)),
        out_shape=jax.ShapeDtypeStruct((S, D), jnp.float32),
        interpret=_INTERPRET,
    )(x, y01, y01, gates, g.reshape(1, D), b.reshape(1, D))


NCHUNK = 16
CT = T // NCHUNK
L = 16


def _sc_mesh():
    return plsc.VectorSubcoreMesh(core_axis_name="c", subcore_axis_name="s")


def _lanes():
    return lax.broadcasted_iota(jnp.int32, (L,), 0)


def _lane_scalar(vec, e):
    return jnp.sum(jnp.where(_lanes() == e, vec, 0))


def _sc_dispatch(ef, x2):
    @functools.partial(
        pl.kernel,
        out_type=(
            jax.ShapeDtypeStruct((PAD, D), jnp.float32),
            jax.ShapeDtypeStruct((NPAIR,), jnp.int32),
            jax.ShapeDtypeStruct((2 * L,), jnp.int32),
        ),
        mesh=_sc_mesh(),
        compiler_params=pltpu.CompilerParams(needs_layout_passes=False),
        scratch_types=[
            pltpu.VMEM((2 * CT,), jnp.int32),
            pltpu.VMEM((CT,), jnp.int32),
            pltpu.VMEM((L,), jnp.int32),
            pltpu.VMEM((NCHUNK * L,), jnp.int32),
            pltpu.VMEM((CT, D), jnp.float32),
            pltpu.VMEM((2 * L,), jnp.int32),
            pltpu.VMEM_SHARED((NCHUNK * L,), jnp.int32),
            pltpu.SemaphoreType.DMA,
        ],
    )
    def k(ef_hbm, x2_hbm, xg_hbm, pos_hbm, be_hbm,
          eids_v, myslots_v, cnt_v, allcnt_v, xr_v, be_v, shc, sem):
        c = lax.axis_index("c")
        s = lax.axis_index("s")
        lanes = _lanes()
        pltpu.sync_copy(ef_hbm.at[pl.ds(s * CT, CT)], eids_v.at[pl.ds(0, CT)])
        pltpu.sync_copy(ef_hbm.at[pl.ds(T + s * CT, CT)],
                        eids_v.at[pl.ds(CT, CT)])
        evs = []
        cs = [jnp.int32(0)] * E
        for v in range(2 * CT // L):
            ev = eids_v[pl.ds(v * L, L)]
            evs.append(ev)
            for e in range(E):
                cs[e] = cs[e] + jnp.sum((ev == e).astype(jnp.int32))
        cnt16 = jnp.zeros((L,), jnp.int32)
        for e in range(E):
            cnt16 = jnp.where(lanes == e, cs[e], cnt16)
        cnt_v[...] = cnt16
        pltpu.sync_copy(cnt_v, shc.at[pl.ds(s * L, L)])
        plsc.subcore_barrier()
        pltpu.sync_copy(shc, allcnt_v)
        totals = jnp.zeros((L,), jnp.int32)
        pre = jnp.zeros((L,), jnp.int32)
        for q in range(NCHUNK):
            row = allcnt_v[pl.ds(q * L, L)]
            totals = totals + row
            pre = pre + jnp.where(jnp.full((L,), q, jnp.int32) < s, row, 0)
        padded = ((totals + (BLK - 1)) // BLK) * BLK
        off = plsc.cumsum(padded) - padded
        mybase = off + pre
        bases = [_lane_scalar(mybase, e) for e in range(E)]
        for v in range(2 * CT // L):
            ev = evs[v]
            p16 = jnp.zeros((L,), jnp.int32)
            for e in range(E):
                m = ev == e
                mi = m.astype(jnp.int32)
                rank = plsc.cumsum(mi) - 1
                p16 = jnp.where(m, bases[e] + rank, p16)
                bases[e] = bases[e] + jnp.sum(mi)
            half = v // (CT // L)
            idx = v % (CT // L)

            @pl.when(c == half)
            def _(p16=p16, idx=idx):
                myslots_v[pl.ds(idx * L, L)] = p16
        pltpu.sync_copy(myslots_v, pos_hbm.at[pl.ds(c * T + s * CT, CT)])
        pltpu.sync_copy(x2_hbm.at[pl.ds(s * CT, CT)], xr_v)
        pltpu.async_copy(xr_v, xg_hbm.at[myslots_v], sem).wait()

        @pl.when((c == 0) & (s == 0))
        def _():
            offs = [_lane_scalar(off, e) for e in range(E)]
            pads = [_lane_scalar(padded, e) for e in range(E)]
            for half in range(2):
                blk = lanes + half * L
                bev = jnp.full((L,), -1, jnp.int32)
                for e in range(E):
                    start = offs[e] // BLK
                    end = (offs[e] + pads[e]) // BLK
                    bev = jnp.where((blk >= start) & (blk < end), e, bev)
                be_v[pl.ds(half * L, L)] = bev
            pltpu.sync_copy(be_v, be_hbm)

    return k(ef, x2)


def _sc_combine(y, pos):
    @functools.partial(
        pl.kernel,
        out_type=jax.ShapeDtypeStruct((2 * T, D), jnp.float32),
        mesh=_sc_mesh(),
        compiler_params=pltpu.CompilerParams(needs_layout_passes=False),
        scratch_types=[
            pltpu.VMEM((CT,), jnp.int32),
            pltpu.VMEM((CT, D), jnp.float32),
            pltpu.SemaphoreType.DMA,
        ],
    )
    def k(y_hbm, pos_hbm, y01_hbm, idx_v, rows_v, sem):
        c = lax.axis_index("c")
        s = lax.axis_index("s")
        pltpu.sync_copy(pos_hbm.at[pl.ds(c * T + s * CT, CT)], idx_v)
        pltpu.async_copy(y_hbm.at[idx_v], rows_v, sem).wait()
        pltpu.sync_copy(rows_v, y01_hbm.at[pl.ds(c * T + s * CT, CT)])

    return k(y, pos)


def _dispatch_jnp(eids):
    ef = eids.T.reshape(NPAIR)
    earange = jnp.arange(E, dtype=jnp.int32)
    counts = jnp.sum((ef[:, None] == earange[None, :]).astype(jnp.int32), axis=0)
    padded = ((counts + BLK - 1) // BLK) * BLK
    off = jnp.concatenate([jnp.zeros((1,), jnp.int32), jnp.cumsum(padded)])[:E]
    rank = jnp.zeros((NPAIR,), jnp.int32)
    for e in range(E):
        msk = ef == e
        csum = jnp.cumsum(msk.astype(jnp.int32)) - 1
        rank = jnp.where(msk, csum, rank)
    pos = off[ef] + rank
    perm = jnp.zeros((PAD,), jnp.int32).at[pos].set(
        jnp.arange(NPAIR, dtype=jnp.int32) % T)
    be = jnp.full((NB,), -1, jnp.int32)
    blk_ids = jnp.arange(NB, dtype=jnp.int32) * BLK
    for e in range(E):
        msk = (blk_ids >= off[e]) & (blk_ids < off[e] + padded[e])
        be = jnp.where(msk, e, be)
    return pos, perm, be


def kernel(tgt, memory, sa_w_in, sa_b_in, sa_w_out, sa_b_out,
           ca_w_in, ca_b_in, ca_w_out, ca_b_out,
           ln1_g, ln1_b, ln2_g, ln2_b, ln3_g, ln3_b,
           router_w, router_b, w1, b1, w2, b2):
    x0 = tgt[0]
    mem = memory[0]

    qkv = _matmul_bias(x0, sa_w_in, sa_b_in)
    ctx = _attention(qkv[:, :D], qkv[:, D:])
    x1 = _proj_res_ln(ctx, sa_w_out, sa_b_out, x0, ln1_g, ln1_b)

    q = _matmul_bias(x1, ca_w_in[:D], ca_b_in[:D])
    kv = _matmul_bias(mem, ca_w_in[D:], ca_b_in[D:])
    ctx2 = _attention(q, kv)
    x2 = _proj_res_ln(ctx2, ca_w_out, ca_b_out, x1, ln2_g, ln2_b)

    eids, gates, aux = _router(x2, router_w, router_b)
    ef = eids.T.reshape(NPAIR)
    xg, pos, be = _sc_dispatch(ef, x2)
    y = _moe_ffn(be, xg, w1, b1, w2, b2)
    y01 = _sc_combine(y, pos)
    xo = _combine_ln(x2, y01, gates, ln3_g, ln3_b)

    return (xo[None], aux.reshape(()))
